# baseline (speedup 1.0000x reference)
import jax
import jax.numpy as jnp
from jax.experimental import pallas as pl


BATCH = 4
SEQ = 2048
DIM = 1024
BS = 256


def _add_kernel(x_ref, t_ref, o_ref):
    o_ref[...] = x_ref[...] + t_ref[...]


def kernel(x, table):
    b, s, d = x.shape
    grid = (s // BS, b)
    return pl.pallas_call(
        _add_kernel,
        grid=grid,
        in_specs=[
            pl.BlockSpec((1, BS, d), lambda j, i: (i, j, 0)),
            pl.BlockSpec((BS, d), lambda j, i: (j, 0)),
        ],
        out_specs=pl.BlockSpec((1, BS, d), lambda j, i: (i, j, 0)),
        out_shape=jax.ShapeDtypeStruct((b, s, d), x.dtype),
    )(x, table)


# speedup vs baseline: 1.4226x; 1.4226x over previous
import jax
import jax.numpy as jnp
from jax.experimental import pallas as pl


BATCH = 4
SEQ = 2048
DIM = 1024
BS = 1024


def _add_kernel(x_ref, t_ref, o_ref):
    o_ref[...] = x_ref[...] + t_ref[...]


def kernel(x, table):
    b, s, d = x.shape
    grid = (s // BS, b)
    return pl.pallas_call(
        _add_kernel,
        grid=grid,
        in_specs=[
            pl.BlockSpec((1, BS, d), lambda j, i: (i, j, 0)),
            pl.BlockSpec((BS, d), lambda j, i: (j, 0)),
        ],
        out_specs=pl.BlockSpec((1, BS, d), lambda j, i: (i, j, 0)),
        out_shape=jax.ShapeDtypeStruct((b, s, d), x.dtype),
    )(x, table)


# speedup vs baseline: 1.5458x; 1.0866x over previous
import jax
import jax.numpy as jnp
from jax.experimental import pallas as pl


BATCH = 4
SEQ = 2048
DIM = 1024
BS = 2048


def _add_kernel(x_ref, t_ref, o_ref):
    o_ref[...] = x_ref[...] + t_ref[...]


def kernel(x, table):
    b, s, d = x.shape
    grid = (s // BS, b)
    return pl.pallas_call(
        _add_kernel,
        grid=grid,
        in_specs=[
            pl.BlockSpec((1, BS, d), lambda j, i: (i, j, 0)),
            pl.BlockSpec((BS, d), lambda j, i: (j, 0)),
        ],
        out_specs=pl.BlockSpec((1, BS, d), lambda j, i: (i, j, 0)),
        out_shape=jax.ShapeDtypeStruct((b, s, d), x.dtype),
    )(x, table)
